# baseline (speedup 1.0000x reference)
import jax
import jax.numpy as jnp
from jax import lax
from jax.experimental import pallas as pl
from jax.experimental.pallas import tpu as pltpu
from jax.experimental.pallas import tpu_sc as plsc

_NUM_CORES = 2
_NUM_SUBCORES = 16
_LANES = 16
_NW = _NUM_CORES * _NUM_SUBCORES

_B = 16384
_D = 16
_BPW = _B // _NW
_NT = 4


def _body(vidx_hbm, hidx_hbm, vt_hbm, vf_hbm, ht_hbm, hf_hbm, wb_hbm,
          out_hbm, *scratch):
    idxv, idxh = scratch[0], scratch[1]
    slots = [scratch[2 + t * _LANES: 2 + (t + 1) * _LANES] for t in range(_NT)]
    wbv = scratch[2 + _NT * _LANES]
    obuf = scratch[3 + _NT * _LANES]
    sem = scratch[4 + _NT * _LANES]

    wid = lax.axis_index("s") * _NUM_CORES + lax.axis_index("c")
    base = wid * _BPW

    pltpu.sync_copy(vidx_hbm.at[pl.ds(base, _BPW)], idxv)
    pltpu.sync_copy(hidx_hbm.at[pl.ds(base, _BPW)], idxh)
    pltpu.sync_copy(wb_hbm, wbv)

    tables = (vt_hbm, vf_hbm, ht_hbm, hf_hbm)
    lane = lax.iota(jnp.int32, _LANES)
    wvec = wbv[pl.ds(0, _LANES)]
    bias_vec = wbv[pl.ds(_LANES, _LANES)]

    def group(g, carry):
        gb = g * _LANES
        iv = idxv[pl.ds(gb, _LANES)]
        ih = idxh[pl.ds(gb, _LANES)]
        scalars = []
        for i in range(_LANES):
            sv = iv[i]
            sh = ih[i]
            scalars.append((sv, sv, sh, sh))
            for t in range(_NT):
                pltpu.async_copy(tables[t].at[scalars[i][t]], slots[t][i], sem)
        for i in range(_LANES):
            for t in range(_NT):
                pltpu.make_async_copy(tables[t].at[scalars[i][t]],
                                      slots[t][i], sem).wait()
        acc = jnp.zeros((_LANES,), jnp.float32)
        for i in range(_LANES):
            rv = (slots[0][i][...] * slots[1][i][...]) * \
                 (slots[2][i][...] * slots[3][i][...]) * wvec
            s = jnp.sum(rv)
            acc = jnp.where(lane == i, s, acc)
        logit = acc + bias_vec
        obuf[pl.ds(gb, _LANES)] = 1.0 / (1.0 + jnp.exp(-logit))
        return carry

    lax.fori_loop(0, _BPW // _LANES, group, 0)
    pltpu.sync_copy(obuf, out_hbm.at[pl.ds(base, _BPW)])


def kernel(v_idxs, h_idxs, virus_table, human_table, vfeats_table,
           hfeats_table, W, b):
    wb = jnp.concatenate([
        W.astype(jnp.float32).reshape(_D),
        jnp.broadcast_to(b.astype(jnp.float32).reshape(1), (_LANES,)),
    ])
    scratch = [
        pltpu.VMEM((_BPW,), jnp.int32),
        pltpu.VMEM((_BPW,), jnp.int32),
    ]
    scratch += [pltpu.VMEM((_LANES,), jnp.float32) for _ in range(_NT * _LANES)]
    scratch += [
        pltpu.VMEM((2 * _LANES,), jnp.float32),
        pltpu.VMEM((_BPW,), jnp.float32),
        pltpu.SemaphoreType.DMA,
    ]
    kfn = pl.kernel(
        _body,
        mesh=plsc.VectorSubcoreMesh(core_axis_name="c", subcore_axis_name="s"),
        out_type=jax.ShapeDtypeStruct((_B,), jnp.float32),
        compiler_params=pltpu.CompilerParams(
            needs_layout_passes=False, use_tc_tiling_on_sc=True),
        scratch_types=scratch,
    )
    out = kfn(v_idxs.astype(jnp.int32), h_idxs.astype(jnp.int32),
              virus_table, vfeats_table, human_table, hfeats_table, wb)
    return out.reshape(_B, 1)


# speedup vs baseline: 1.3845x; 1.3845x over previous
import jax
import jax.numpy as jnp
from jax import lax
from jax.experimental import pallas as pl
from jax.experimental.pallas import tpu as pltpu
from jax.experimental.pallas import tpu_sc as plsc

_NUM_CORES = 2
_NUM_SUBCORES = 16
_LANES = 16
_NW = _NUM_CORES * _NUM_SUBCORES

_B = 16384
_D = 16
_BPW = _B // _NW
_NT = 4
_TBLK = 12800


def _tr_body(a_ref, b_ref, oa_ref, ob_ref):
    oa_ref[...] = a_ref[...].T
    ob_ref[...] = b_ref[...].T


def _transpose_pair(at, bt):
    n = at.shape[1]
    grid = (n + _TBLK - 1) // _TBLK
    return pl.pallas_call(
        _tr_body,
        grid=(grid,),
        in_specs=[
            pl.BlockSpec((_D, _TBLK), lambda i: (0, i)),
            pl.BlockSpec((_D, _TBLK), lambda i: (0, i)),
        ],
        out_specs=[
            pl.BlockSpec((_TBLK, _D), lambda i: (i, 0)),
            pl.BlockSpec((_TBLK, _D), lambda i: (i, 0)),
        ],
        out_shape=[
            jax.ShapeDtypeStruct((n, _D), jnp.float32),
            jax.ShapeDtypeStruct((n, _D), jnp.float32),
        ],
    )(at, bt)


def _body(vidx_hbm, hidx_hbm, vt_hbm, vf_hbm, ht_hbm, hf_hbm, wb_hbm,
          out_hbm, *scratch):
    idxv, idxh = scratch[0], scratch[1]
    slots = [scratch[2 + t * _LANES: 2 + (t + 1) * _LANES] for t in range(_NT)]
    wbv = scratch[2 + _NT * _LANES]
    obuf = scratch[3 + _NT * _LANES]
    sem = scratch[4 + _NT * _LANES]

    wid = lax.axis_index("s") * _NUM_CORES + lax.axis_index("c")
    base = wid * _BPW

    pltpu.sync_copy(vidx_hbm.at[pl.ds(base, _BPW)], idxv)
    pltpu.sync_copy(hidx_hbm.at[pl.ds(base, _BPW)], idxh)
    pltpu.sync_copy(wb_hbm, wbv)

    tables = (vt_hbm, vf_hbm, ht_hbm, hf_hbm)
    lane = lax.iota(jnp.int32, _LANES)
    wvec = wbv[pl.ds(0, _LANES)]
    bias_vec = wbv[pl.ds(_LANES, _LANES)]

    def group(g, carry):
        gb = g * _LANES
        iv = idxv[pl.ds(gb, _LANES)]
        ih = idxh[pl.ds(gb, _LANES)]
        scalars = []
        for i in range(_LANES):
            sv = iv[i]
            sh = ih[i]
            scalars.append((sv, sv, sh, sh))
            for t in range(_NT):
                pltpu.async_copy(tables[t].at[scalars[i][t]], slots[t][i], sem)
        for i in range(_LANES):
            for t in range(_NT):
                pltpu.make_async_copy(tables[t].at[scalars[i][t]],
                                      slots[t][i], sem).wait()
        acc = jnp.zeros((_LANES,), jnp.float32)
        for i in range(_LANES):
            rv = (slots[0][i][...] * slots[1][i][...]) * \
                 (slots[2][i][...] * slots[3][i][...]) * wvec
            s = jnp.sum(rv)
            acc = jnp.where(lane == i, s, acc)
        logit = acc + bias_vec
        obuf[pl.ds(gb, _LANES)] = 1.0 / (1.0 + jnp.exp(-logit))
        return carry

    lax.fori_loop(0, _BPW // _LANES, group, 0)
    pltpu.sync_copy(obuf, out_hbm.at[pl.ds(base, _BPW)])


def kernel(v_idxs, h_idxs, virus_table, human_table, vfeats_table,
           hfeats_table, W, b):
    wb = jnp.concatenate([
        W.astype(jnp.float32).reshape(_D),
        jnp.broadcast_to(b.astype(jnp.float32).reshape(1), (_LANES,)),
    ])
    vt2, vf2 = _transpose_pair(virus_table.T, vfeats_table.T)
    ht2, hf2 = _transpose_pair(human_table.T, hfeats_table.T)
    scratch = [
        pltpu.VMEM((_BPW,), jnp.int32),
        pltpu.VMEM((_BPW,), jnp.int32),
    ]
    scratch += [pltpu.VMEM((_LANES,), jnp.float32) for _ in range(_NT * _LANES)]
    scratch += [
        pltpu.VMEM((2 * _LANES,), jnp.float32),
        pltpu.VMEM((_BPW,), jnp.float32),
        pltpu.SemaphoreType.DMA,
    ]
    kfn = pl.kernel(
        _body,
        mesh=plsc.VectorSubcoreMesh(core_axis_name="c", subcore_axis_name="s"),
        out_type=jax.ShapeDtypeStruct((_B,), jnp.float32),
        compiler_params=pltpu.CompilerParams(
            needs_layout_passes=False, use_tc_tiling_on_sc=True),
        scratch_types=scratch,
    )
    out = kfn(v_idxs.astype(jnp.int32), h_idxs.astype(jnp.int32),
              vt2, vf2, ht2, hf2, wb)
    return out.reshape(_B, 1)


# speedup vs baseline: 2.1855x; 1.5785x over previous
import jax
import jax.numpy as jnp
from jax import lax
from jax.experimental import pallas as pl
from jax.experimental.pallas import tpu as pltpu
from jax.experimental.pallas import tpu_sc as plsc

_NUM_CORES = 2
_NUM_SUBCORES = 16
_LANES = 16
_NW = _NUM_CORES * _NUM_SUBCORES

_B = 16384
_D = 16
_BPW = _B // _NW
_NT = 2
_TBLK = 12800


def _prod_tr_body(a_ref, b_ref, o_ref):
    o_ref[...] = (a_ref[...] * b_ref[...]).T


def _product_table(at, bt):
    n = at.shape[1]
    grid = (n + _TBLK - 1) // _TBLK
    return pl.pallas_call(
        _prod_tr_body,
        grid=(grid,),
        in_specs=[
            pl.BlockSpec((_D, _TBLK), lambda i: (0, i)),
            pl.BlockSpec((_D, _TBLK), lambda i: (0, i)),
        ],
        out_specs=pl.BlockSpec((_TBLK, _D), lambda i: (i, 0)),
        out_shape=jax.ShapeDtypeStruct((n, _D), jnp.float32),
    )(at, bt)


def _body(vidx_hbm, hidx_hbm, vp_hbm, hp_hbm, wb_hbm, out_hbm, *scratch):
    idxv, idxh = scratch[0], scratch[1]
    slots = [scratch[2 + t * _LANES: 2 + (t + 1) * _LANES] for t in range(_NT)]
    wbv = scratch[2 + _NT * _LANES]
    obuf = scratch[3 + _NT * _LANES]
    sem = scratch[4 + _NT * _LANES]

    wid = lax.axis_index("s") * _NUM_CORES + lax.axis_index("c")
    base = wid * _BPW

    pltpu.sync_copy(vidx_hbm.at[pl.ds(base, _BPW)], idxv)
    pltpu.sync_copy(hidx_hbm.at[pl.ds(base, _BPW)], idxh)
    pltpu.sync_copy(wb_hbm, wbv)

    tables = (vp_hbm, hp_hbm)
    lane = lax.iota(jnp.int32, _LANES)
    wvec = wbv[pl.ds(0, _LANES)]
    bias_vec = wbv[pl.ds(_LANES, _LANES)]

    def group(g, carry):
        gb = g * _LANES
        iv = idxv[pl.ds(gb, _LANES)]
        ih = idxh[pl.ds(gb, _LANES)]
        scalars = []
        for i in range(_LANES):
            scalars.append((iv[i], ih[i]))
            for t in range(_NT):
                pltpu.async_copy(tables[t].at[scalars[i][t]], slots[t][i], sem)
        for i in range(_LANES):
            for t in range(_NT):
                pltpu.make_async_copy(tables[t].at[scalars[i][t]],
                                      slots[t][i], sem).wait()
        acc = jnp.zeros((_LANES,), jnp.float32)
        for i in range(_LANES):
            rv = slots[0][i][...] * slots[1][i][...] * wvec
            s = jnp.sum(rv)
            acc = jnp.where(lane == i, s, acc)
        logit = acc + bias_vec
        obuf[pl.ds(gb, _LANES)] = 1.0 / (1.0 + jnp.exp(-logit))
        return carry

    lax.fori_loop(0, _BPW // _LANES, group, 0)
    pltpu.sync_copy(obuf, out_hbm.at[pl.ds(base, _BPW)])


def kernel(v_idxs, h_idxs, virus_table, human_table, vfeats_table,
           hfeats_table, W, b):
    wb = jnp.concatenate([
        W.astype(jnp.float32).reshape(_D),
        jnp.broadcast_to(b.astype(jnp.float32).reshape(1), (_LANES,)),
    ])
    vp = _product_table(virus_table.T, vfeats_table.T)
    hp = _product_table(human_table.T, hfeats_table.T)
    scratch = [
        pltpu.VMEM((_BPW,), jnp.int32),
        pltpu.VMEM((_BPW,), jnp.int32),
    ]
    scratch += [pltpu.VMEM((_LANES,), jnp.float32) for _ in range(_NT * _LANES)]
    scratch += [
        pltpu.VMEM((2 * _LANES,), jnp.float32),
        pltpu.VMEM((_BPW,), jnp.float32),
        pltpu.SemaphoreType.DMA,
    ]
    kfn = pl.kernel(
        _body,
        mesh=plsc.VectorSubcoreMesh(core_axis_name="c", subcore_axis_name="s"),
        out_type=jax.ShapeDtypeStruct((_B,), jnp.float32),
        compiler_params=pltpu.CompilerParams(
            needs_layout_passes=False, use_tc_tiling_on_sc=True),
        scratch_types=scratch,
    )
    out = kfn(v_idxs.astype(jnp.int32), h_idxs.astype(jnp.int32), vp, hp, wb)
    return out.reshape(_B, 1)


# speedup vs baseline: 2.2703x; 1.0388x over previous
import jax
import jax.numpy as jnp
from jax import lax
from jax.experimental import pallas as pl
from jax.experimental.pallas import tpu as pltpu
from jax.experimental.pallas import tpu_sc as plsc

_NUM_CORES = 2
_NUM_SUBCORES = 16
_LANES = 16
_NW = _NUM_CORES * _NUM_SUBCORES

_B = 16384
_D = 16
_BPW = _B // _NW
_NT = 2
_TBLK = 25600


def _prod_tr_body(a_ref, b_ref, o_ref):
    o_ref[...] = (a_ref[...] * b_ref[...]).T


def _product_table(at, bt):
    n = at.shape[1]
    grid = (n + _TBLK - 1) // _TBLK
    return pl.pallas_call(
        _prod_tr_body,
        grid=(grid,),
        in_specs=[
            pl.BlockSpec((_D, _TBLK), lambda i: (0, i)),
            pl.BlockSpec((_D, _TBLK), lambda i: (0, i)),
        ],
        out_specs=pl.BlockSpec((_TBLK, _D), lambda i: (i, 0)),
        out_shape=jax.ShapeDtypeStruct((n, _D), jnp.float32),
    )(at, bt)


def _body(vidx_hbm, hidx_hbm, vp_hbm, hp_hbm, wb_hbm, out_hbm, *scratch):
    idxv, idxh = scratch[0], scratch[1]
    slots = [scratch[2 + t * _LANES: 2 + (t + 1) * _LANES] for t in range(_NT)]
    wbv = scratch[2 + _NT * _LANES]
    obuf = scratch[3 + _NT * _LANES]
    sem = scratch[4 + _NT * _LANES]

    wid = lax.axis_index("s") * _NUM_CORES + lax.axis_index("c")
    base = wid * _BPW

    pltpu.sync_copy(vidx_hbm.at[pl.ds(base, _BPW)], idxv)
    pltpu.sync_copy(hidx_hbm.at[pl.ds(base, _BPW)], idxh)
    pltpu.sync_copy(wb_hbm, wbv)

    tables = (vp_hbm, hp_hbm)
    lane = lax.iota(jnp.int32, _LANES)
    wvec = wbv[pl.ds(0, _LANES)]
    bias_vec = wbv[pl.ds(_LANES, _LANES)]

    def group(g, carry):
        gb = g * _LANES
        iv = idxv[pl.ds(gb, _LANES)]
        ih = idxh[pl.ds(gb, _LANES)]
        scalars = []
        for i in range(_LANES):
            scalars.append((iv[i], ih[i]))
            for t in range(_NT):
                pltpu.async_copy(tables[t].at[scalars[i][t]], slots[t][i], sem)
        for i in range(_LANES):
            for t in range(_NT):
                pltpu.make_async_copy(tables[t].at[scalars[i][t]],
                                      slots[t][i], sem).wait()
        acc = jnp.zeros((_LANES,), jnp.float32)
        for i in range(_LANES):
            rv = slots[0][i][...] * slots[1][i][...] * wvec
            s = jnp.sum(rv)
            acc = jnp.where(lane == i, s, acc)
        logit = acc + bias_vec
        obuf[pl.ds(gb, _LANES)] = 1.0 / (1.0 + jnp.exp(-logit))
        return carry

    lax.fori_loop(0, _BPW // _LANES, group, 0)
    pltpu.sync_copy(obuf, out_hbm.at[pl.ds(base, _BPW)])


def kernel(v_idxs, h_idxs, virus_table, human_table, vfeats_table,
           hfeats_table, W, b):
    wb = jnp.concatenate([
        W.astype(jnp.float32).reshape(_D),
        jnp.broadcast_to(b.astype(jnp.float32).reshape(1), (_LANES,)),
    ])
    vp = _product_table(virus_table.T, vfeats_table.T)
    hp = _product_table(human_table.T, hfeats_table.T)
    scratch = [
        pltpu.VMEM((_BPW,), jnp.int32),
        pltpu.VMEM((_BPW,), jnp.int32),
    ]
    scratch += [pltpu.VMEM((_LANES,), jnp.float32) for _ in range(_NT * _LANES)]
    scratch += [
        pltpu.VMEM((2 * _LANES,), jnp.float32),
        pltpu.VMEM((_BPW,), jnp.float32),
        pltpu.SemaphoreType.DMA,
    ]
    kfn = pl.kernel(
        _body,
        mesh=plsc.VectorSubcoreMesh(core_axis_name="c", subcore_axis_name="s"),
        out_type=jax.ShapeDtypeStruct((_B,), jnp.float32),
        compiler_params=pltpu.CompilerParams(
            needs_layout_passes=False, use_tc_tiling_on_sc=True),
        scratch_types=scratch,
    )
    out = kfn(v_idxs.astype(jnp.int32), h_idxs.astype(jnp.int32), vp, hp, wb)
    return out.reshape(_B, 1)


# speedup vs baseline: 2.2796x; 1.0041x over previous
import jax
import jax.numpy as jnp
from jax import lax
from jax.experimental import pallas as pl
from jax.experimental.pallas import tpu as pltpu
from jax.experimental.pallas import tpu_sc as plsc

_NUM_CORES = 2
_NUM_SUBCORES = 16
_LANES = 16
_NW = _NUM_CORES * _NUM_SUBCORES

_B = 16384
_D = 16
_BPW = _B // _NW
_NT = 2
_TBLK = 32000


def _prod_tr_body(a_ref, b_ref, o_ref):
    o_ref[...] = (a_ref[...] * b_ref[...]).T


def _product_table(at, bt):
    n = at.shape[1]
    grid = (n + _TBLK - 1) // _TBLK
    return pl.pallas_call(
        _prod_tr_body,
        grid=(grid,),
        in_specs=[
            pl.BlockSpec((_D, _TBLK), lambda i: (0, i)),
            pl.BlockSpec((_D, _TBLK), lambda i: (0, i)),
        ],
        out_specs=pl.BlockSpec((_TBLK, _D), lambda i: (i, 0)),
        out_shape=jax.ShapeDtypeStruct((n, _D), jnp.float32),
    )(at, bt)


def _body(vidx_hbm, hidx_hbm, vp_hbm, hp_hbm, wb_hbm, out_hbm, *scratch):
    idxv, idxh = scratch[0], scratch[1]
    slots = [scratch[2 + t * _LANES: 2 + (t + 1) * _LANES] for t in range(_NT)]
    wbv = scratch[2 + _NT * _LANES]
    obuf = scratch[3 + _NT * _LANES]
    sem = scratch[4 + _NT * _LANES]

    wid = lax.axis_index("s") * _NUM_CORES + lax.axis_index("c")
    base = wid * _BPW

    pltpu.sync_copy(vidx_hbm.at[pl.ds(base, _BPW)], idxv)
    pltpu.sync_copy(hidx_hbm.at[pl.ds(base, _BPW)], idxh)
    pltpu.sync_copy(wb_hbm, wbv)

    tables = (vp_hbm, hp_hbm)
    lane = lax.iota(jnp.int32, _LANES)
    wvec = wbv[pl.ds(0, _LANES)]
    bias_vec = wbv[pl.ds(_LANES, _LANES)]

    def group(g, carry):
        gb = g * _LANES
        iv = idxv[pl.ds(gb, _LANES)]
        ih = idxh[pl.ds(gb, _LANES)]
        scalars = []
        for i in range(_LANES):
            scalars.append((iv[i], ih[i]))
            for t in range(_NT):
                pltpu.async_copy(tables[t].at[scalars[i][t]], slots[t][i], sem)
        for i in range(_LANES):
            for t in range(_NT):
                pltpu.make_async_copy(tables[t].at[scalars[i][t]],
                                      slots[t][i], sem).wait()
        acc = jnp.zeros((_LANES,), jnp.float32)
        for i in range(_LANES):
            rv = slots[0][i][...] * slots[1][i][...] * wvec
            s = jnp.sum(rv)
            acc = jnp.where(lane == i, s, acc)
        logit = acc + bias_vec
        obuf[pl.ds(gb, _LANES)] = 1.0 / (1.0 + jnp.exp(-logit))
        return carry

    lax.fori_loop(0, _BPW // _LANES, group, 0)
    pltpu.sync_copy(obuf, out_hbm.at[pl.ds(base, _BPW)])


def kernel(v_idxs, h_idxs, virus_table, human_table, vfeats_table,
           hfeats_table, W, b):
    wb = jnp.concatenate([
        W.astype(jnp.float32).reshape(_D),
        jnp.broadcast_to(b.astype(jnp.float32).reshape(1), (_LANES,)),
    ])
    vp = _product_table(virus_table.T, vfeats_table.T)
    hp = _product_table(human_table.T, hfeats_table.T)
    scratch = [
        pltpu.VMEM((_BPW,), jnp.int32),
        pltpu.VMEM((_BPW,), jnp.int32),
    ]
    scratch += [pltpu.VMEM((_LANES,), jnp.float32) for _ in range(_NT * _LANES)]
    scratch += [
        pltpu.VMEM((2 * _LANES,), jnp.float32),
        pltpu.VMEM((_BPW,), jnp.float32),
        pltpu.SemaphoreType.DMA,
    ]
    kfn = pl.kernel(
        _body,
        mesh=plsc.VectorSubcoreMesh(core_axis_name="c", subcore_axis_name="s"),
        out_type=jax.ShapeDtypeStruct((_B,), jnp.float32),
        compiler_params=pltpu.CompilerParams(
            needs_layout_passes=False, use_tc_tiling_on_sc=True),
        scratch_types=scratch,
    )
    out = kfn(v_idxs.astype(jnp.int32), h_idxs.astype(jnp.int32), vp, hp, wb)
    return out.reshape(_B, 1)
